# Initial kernel scaffold; baseline (speedup 1.0000x reference)
import jax
import jax.numpy as jnp
from jax import lax
from jax.experimental import pallas as pl
from jax.experimental.pallas import tpu as pltpu
from jax.experimental.pallas import tpu_sc as plsc

N, E, D = 10000, 320000, 128

NC = 2
NS = 16
NW = NC * NS
EPW = E // NW
B = 100
NJ = EPW // B
RPS = N // NS


def _linear_body(x_ref, w_ref, b_ref, o_ref):
    o_ref[...] = lax.dot_general(
        x_ref[...], w_ref[...],
        dimension_numbers=(((1,), (1,)), ((), ())),
        preferred_element_type=jnp.float32,
    ) + b_ref[...]


def _linear(x, W, b):
    blk = 1000
    return pl.pallas_call(
        _linear_body,
        grid=(N // blk,),
        in_specs=[
            pl.BlockSpec((blk, D), lambda i: (i, 0)),
            pl.BlockSpec((D, D), lambda i: (0, 0)),
            pl.BlockSpec((1, D), lambda i: (0, 0)),
        ],
        out_specs=pl.BlockSpec((blk, D), lambda i: (i, 0)),
        out_shape=jax.ShapeDtypeStruct((N, D), jnp.float32),
    )(x, W, b.reshape(1, D))


def _sc_body(support_hbm, src_hbm, dst_hbm, w_hbm, zeros_hbm, out_hbm,
             srcbuf, dstbuf, wbuf, gbuf_a, gbuf_b, acc,
             sem_ga, sem_gb, sem_i):
    cid = lax.axis_index("c")
    sid = lax.axis_index("s")
    wid = sid * NC + cid
    row0 = wid * NJ

    pltpu.sync_copy(src_hbm.at[pl.ds(row0, NJ)], srcbuf)
    pltpu.sync_copy(dst_hbm.at[pl.ds(row0, NJ)], dstbuf)
    pltpu.sync_copy(w_hbm.at[pl.ds(wid * EPW, EPW)], wbuf)

    pltpu.async_copy(zeros_hbm.at[pl.ds(sid * RPS, RPS)],
                     acc.at[pl.ds(sid * RPS, RPS)], sem_i).wait()
    plsc.subcore_barrier()

    pltpu.async_copy(support_hbm.at[srcbuf.at[0]], gbuf_a, sem_ga)
    pltpu.async_copy(support_hbm.at[srcbuf.at[1]], gbuf_b, sem_gb)

    @pl.loop(0, NJ, step=2)
    def _(j):
        for off, gbuf, sem in ((0, gbuf_a, sem_ga), (1, gbuf_b, sem_gb)):
            jj = j + off
            pltpu.make_async_copy(support_hbm.at[srcbuf.at[jj]], gbuf,
                                  sem).wait()

            @pl.loop(0, B)
            def _(e):
                wsplat = plsc.load_gather(
                    wbuf, [jnp.full((16,), jj * B + e, jnp.int32)])
                for c in range(D // 16):
                    sl = pl.ds(c * 16, 16)
                    gbuf[e, sl] = gbuf[e, sl] * wsplat

            pltpu.sync_copy(gbuf, acc.at[dstbuf.at[jj]], add=True)

            @pl.when(jj + 2 < NJ)
            def _():
                pltpu.async_copy(support_hbm.at[srcbuf.at[jj + 2]], gbuf,
                                 sem)

    plsc.subcore_barrier()
    pltpu.sync_copy(acc.at[pl.ds(sid * RPS, RPS)],
                    out_hbm.at[cid, pl.ds(sid * RPS, RPS)])


def _sc_scatter(support, src, dst, w, zeros):
    mesh = plsc.VectorSubcoreMesh(core_axis_name="c", subcore_axis_name="s")
    kern = pl.kernel(
        _sc_body,
        out_type=jax.ShapeDtypeStruct((NC, N, D), jnp.float32),
        mesh=mesh,
        scratch_types=[
            pltpu.VMEM((NJ, B), jnp.int32),
            pltpu.VMEM((NJ, B), jnp.int32),
            pltpu.VMEM((EPW,), jnp.float32),
            pltpu.VMEM((B, D), jnp.float32),
            pltpu.VMEM((B, D), jnp.float32),
            pltpu.VMEM_SHARED((N, D), jnp.float32),
            pltpu.SemaphoreType.DMA,
            pltpu.SemaphoreType.DMA,
            pltpu.SemaphoreType.DMA,
        ],
    )
    return kern(support, src, dst, w, zeros)


def _combine_body(p_ref, o_ref):
    o_ref[...] = p_ref[0] + p_ref[1]


def _combine(partials):
    blk = 1000
    return pl.pallas_call(
        _combine_body,
        grid=(N // blk,),
        in_specs=[pl.BlockSpec((NC, blk, D), lambda i: (0, i, 0))],
        out_specs=pl.BlockSpec((blk, D), lambda i: (i, 0)),
        out_shape=jax.ShapeDtypeStruct((N, D), jnp.float32),
    )(partials)


@jax.jit
def _impl(x, edge_index, edge_weight, W, b):
    support = _linear(x, W, b)
    src = edge_index[0].astype(jnp.int32).reshape(E // B, B)
    dst = edge_index[1].astype(jnp.int32).reshape(E // B, B)
    zeros = jnp.zeros((N, D), jnp.float32)
    partials = _sc_scatter(support, src, dst, edge_weight, zeros)
    return _combine(partials)


def kernel(x, edge_index, edge_weight, W, b):
    return _impl(x, edge_index, edge_weight, W, b)


# speedup vs baseline: 7.3879x; 7.3879x over previous
import dataclasses

import jax
import jax.numpy as jnp
from jax import lax
from jax.experimental import pallas as pl
from jax.experimental.pallas import tpu as pltpu
from jax.experimental.pallas import tpu_sc as plsc

N, E, D = 10000, 320000, 128

NC = 2
NS = 16
NW = NC * NS
EPW = E // NW
NST = 2
B = 50
NJS = EPW // (NST * B)
EPS = NJS * B
RPS = 632
NP = NS * RPS


def _linear_body(x_ref, w_ref, b_ref, o_ref):
    o_ref[...] = lax.dot_general(
        x_ref[...], w_ref[...],
        dimension_numbers=(((1,), (1,)), ((), ())),
        preferred_element_type=jnp.float32,
    ) + b_ref[...]


def _linear(x, W, b):
    blk = 1000
    return pl.pallas_call(
        _linear_body,
        grid=(N // blk,),
        in_specs=[
            pl.BlockSpec((blk, D), lambda i: (i, 0)),
            pl.BlockSpec((D, D), lambda i: (0, 0)),
            pl.BlockSpec((1, D), lambda i: (0, 0)),
        ],
        out_specs=pl.BlockSpec((blk, D), lambda i: (i, 0)),
        out_shape=jax.ShapeDtypeStruct((N, D), jnp.float32),
    )(x, W, b.reshape(1, D))


def _sc_body(support_hbm, src_hbm, dst_hbm, w_hbm, zeros_hbm, out_hbm,
             srcbuf, dstbuf, wbuf, gbuf_a, gbuf_b, acc,
             sem_ga, sem_gb, sem_i):
    cid = lax.axis_index("c")
    sid = lax.axis_index("s")
    wid = sid * NC + cid

    pltpu.async_copy(zeros_hbm.at[pl.ds(sid * RPS, RPS)],
                     acc.at[pl.ds(sid * RPS, RPS)], sem_i).wait()
    plsc.subcore_barrier()

    for s in range(NST):
        pltpu.sync_copy(src_hbm.at[wid, s], srcbuf)
        pltpu.sync_copy(dst_hbm.at[wid, s], dstbuf)
        pltpu.sync_copy(w_hbm.at[pl.ds(wid * EPW + s * EPS, EPS)], wbuf)

        pltpu.async_copy(support_hbm.at[srcbuf.at[0]], gbuf_a, sem_ga)
        pltpu.async_copy(support_hbm.at[srcbuf.at[1]], gbuf_b, sem_gb)

        @pl.loop(0, NJS, step=2)
        def _(j):
            for off, gbuf, sem in ((0, gbuf_a, sem_ga), (1, gbuf_b, sem_gb)):
                jj = j + off
                pltpu.make_async_copy(support_hbm.at[srcbuf.at[jj]], gbuf,
                                      sem).wait()

                @pl.loop(0, B)
                def _(e):
                    wsplat = plsc.load_gather(
                        wbuf, [jnp.full((16,), jj * B + e, jnp.int32)])
                    for c in range(D // 16):
                        sl = pl.ds(c * 16, 16)
                        gbuf[e, sl] = gbuf[e, sl] * wsplat

                pltpu.sync_copy(gbuf, acc.at[dstbuf.at[jj]], add=True)

                @pl.when(jj + 2 < NJS)
                def _():
                    pltpu.async_copy(support_hbm.at[srcbuf.at[jj + 2]],
                                     gbuf, sem)

    plsc.subcore_barrier()
    pltpu.sync_copy(acc.at[pl.ds(sid * RPS, RPS)],
                    out_hbm.at[cid, pl.ds(sid * RPS, RPS)])


def _sc_scatter(support, src, dst, w, zeros):
    mesh = plsc.VectorSubcoreMesh(core_axis_name="c", subcore_axis_name="s")
    cp = pltpu.CompilerParams()
    if "needs_layout_passes" in pltpu.CompilerParams.__dataclass_fields__:
        cp = dataclasses.replace(cp, needs_layout_passes=False)
    kern = pl.kernel(
        _sc_body,
        compiler_params=cp,
        out_type=jax.ShapeDtypeStruct((NC, NP, D), jnp.float32),
        mesh=mesh,
        scratch_types=[
            pltpu.VMEM((NJS, B), jnp.int32),
            pltpu.VMEM((NJS, B), jnp.int32),
            pltpu.VMEM((EPS,), jnp.float32),
            pltpu.VMEM((B, D), jnp.float32),
            pltpu.VMEM((B, D), jnp.float32),
            pltpu.VMEM_SHARED((NP, D), jnp.float32),
            pltpu.SemaphoreType.DMA,
            pltpu.SemaphoreType.DMA,
            pltpu.SemaphoreType.DMA,
        ],
    )
    return kern(support, src, dst, w, zeros)


def _combine_body(p_ref, o_ref):
    o_ref[...] = p_ref[0] + p_ref[1]


def _combine(partials):
    blk = 1000
    return pl.pallas_call(
        _combine_body,
        grid=(N // blk,),
        in_specs=[pl.BlockSpec((NC, blk, D), lambda i: (0, i, 0))],
        out_specs=pl.BlockSpec((blk, D), lambda i: (i, 0)),
        out_shape=jax.ShapeDtypeStruct((N, D), jnp.float32),
    )(partials)


@jax.jit
def _impl(x, edge_index, edge_weight, W, b):
    support = _linear(x, W, b)
    src = edge_index[0].astype(jnp.int32).reshape(NW, NST, NJS, B)
    dst = edge_index[1].astype(jnp.int32).reshape(NW, NST, NJS, B)
    zeros = jnp.zeros((NP, D), jnp.float32)
    partials = _sc_scatter(support, src, dst, edge_weight, zeros)
    return _combine(partials)


def kernel(x, edge_index, edge_weight, W, b):
    return _impl(x, edge_index, edge_weight, W, b)


# speedup vs baseline: 9.1962x; 1.2448x over previous
import dataclasses

import jax
import jax.numpy as jnp
from jax import lax
from jax.experimental import pallas as pl
from jax.experimental.pallas import tpu as pltpu
from jax.experimental.pallas import tpu_sc as plsc

N, E, D = 10000, 320000, 128

NC = 2
NS = 16
NW = NC * NS
EPW = E // NW
NST = 5
B = 100
NJS = EPW // (NST * B)
EPS = NJS * B
RPS = 632
NP = NS * RPS


def _linear_body(x_ref, w_ref, b_ref, o_ref):
    o_ref[...] = lax.dot_general(
        x_ref[...], w_ref[...],
        dimension_numbers=(((1,), (1,)), ((), ())),
        preferred_element_type=jnp.float32,
    ) + b_ref[...]


def _linear(x, W, b):
    blk = 1000
    return pl.pallas_call(
        _linear_body,
        grid=(N // blk,),
        in_specs=[
            pl.BlockSpec((blk, D), lambda i: (i, 0)),
            pl.BlockSpec((D, D), lambda i: (0, 0)),
            pl.BlockSpec((1, D), lambda i: (0, 0)),
        ],
        out_specs=pl.BlockSpec((blk, D), lambda i: (i, 0)),
        out_shape=jax.ShapeDtypeStruct((N, D), jnp.float32),
    )(x, W, b.reshape(1, D))


def _sc_body(support_hbm, src_hbm, dst_hbm, w_hbm, zeros_hbm, out_hbm,
             srcbuf, dstbuf, wbuf, gbuf_a, gbuf_b, acc,
             sem_ga, sem_gb, sem_i):
    cid = lax.axis_index("c")
    sid = lax.axis_index("s")
    wid = sid * NC + cid

    pltpu.async_copy(zeros_hbm.at[pl.ds(sid * RPS, RPS)],
                     acc.at[pl.ds(sid * RPS, RPS)], sem_i).wait()
    plsc.subcore_barrier()

    for s in range(NST):
        pltpu.sync_copy(src_hbm.at[wid, s], srcbuf)
        pltpu.sync_copy(dst_hbm.at[wid, s], dstbuf)
        pltpu.sync_copy(w_hbm.at[pl.ds(wid * EPW + s * EPS, EPS)], wbuf)

        pltpu.async_copy(support_hbm.at[srcbuf.at[0]], gbuf_a, sem_ga)
        pltpu.async_copy(support_hbm.at[srcbuf.at[1]], gbuf_b, sem_gb)

        @pl.loop(0, NJS, step=2)
        def _(j):
            for off, gbuf, sem in ((0, gbuf_a, sem_ga), (1, gbuf_b, sem_gb)):
                jj = j + off
                pltpu.make_async_copy(support_hbm.at[srcbuf.at[jj]], gbuf,
                                      sem).wait()

                @plsc.parallel_loop(0, B, unroll=4)
                def _(e):
                    wsplat = plsc.load_gather(
                        wbuf, [jnp.full((16,), jj * B + e, jnp.int32)])
                    for c in range(D // 16):
                        sl = pl.ds(c * 16, 16)
                        gbuf[e, sl] = gbuf[e, sl] * wsplat

                pltpu.sync_copy(gbuf, acc.at[dstbuf.at[jj]], add=True)

                @pl.when(jj + 2 < NJS)
                def _():
                    pltpu.async_copy(support_hbm.at[srcbuf.at[jj + 2]],
                                     gbuf, sem)

    plsc.subcore_barrier()
    pltpu.sync_copy(acc.at[pl.ds(sid * RPS, RPS)],
                    out_hbm.at[cid, pl.ds(sid * RPS, RPS)])


def _sc_scatter(support, src, dst, w, zeros):
    mesh = plsc.VectorSubcoreMesh(core_axis_name="c", subcore_axis_name="s")
    cp = pltpu.CompilerParams()
    if "needs_layout_passes" in pltpu.CompilerParams.__dataclass_fields__:
        cp = dataclasses.replace(cp, needs_layout_passes=False)
    kern = pl.kernel(
        _sc_body,
        compiler_params=cp,
        out_type=jax.ShapeDtypeStruct((NC, NP, D), jnp.float32),
        mesh=mesh,
        scratch_types=[
            pltpu.VMEM((NJS, B), jnp.int32),
            pltpu.VMEM((NJS, B), jnp.int32),
            pltpu.VMEM((EPS,), jnp.float32),
            pltpu.VMEM((B, D), jnp.float32),
            pltpu.VMEM((B, D), jnp.float32),
            pltpu.VMEM_SHARED((NP, D), jnp.float32),
            pltpu.SemaphoreType.DMA,
            pltpu.SemaphoreType.DMA,
            pltpu.SemaphoreType.DMA,
        ],
    )
    return kern(support, src, dst, w, zeros)


def _combine_body(p_ref, o_ref):
    o_ref[...] = p_ref[0] + p_ref[1]


def _combine(partials):
    blk = 1000
    return pl.pallas_call(
        _combine_body,
        grid=(N // blk,),
        in_specs=[pl.BlockSpec((NC, blk, D), lambda i: (0, i, 0))],
        out_specs=pl.BlockSpec((blk, D), lambda i: (i, 0)),
        out_shape=jax.ShapeDtypeStruct((N, D), jnp.float32),
    )(partials)


@jax.jit
def _impl(x, edge_index, edge_weight, W, b):
    support = _linear(x, W, b)
    src = edge_index[0].astype(jnp.int32).reshape(NW, NST, NJS, B)
    dst = edge_index[1].astype(jnp.int32).reshape(NW, NST, NJS, B)
    zeros = jnp.zeros((NP, D), jnp.float32)
    partials = _sc_scatter(support, src, dst, edge_weight, zeros)
    return _combine(partials)


def kernel(x, edge_index, edge_weight, W, b):
    return _impl(x, edge_index, edge_weight, W, b)
